# baseline (speedup 1.0000x reference)
import functools

import jax
import jax.numpy as jnp
from jax import lax
from jax.experimental import pallas as pl
from jax.experimental.pallas import tpu as pltpu
from jax.experimental.pallas import tpu_sc as plsc

B = 16384
F = 26
V = 100000
D = 16
NUM = 13
OW = NUM + F * D
OWP = 432

L = 16
NC, NS = 2, 16
NW = NC * NS
COLS = F * D
CPW = COLS // NW
SL = 128
NST = B // SL


@functools.partial(
    pl.kernel,
    out_type=jax.ShapeDtypeStruct((OWP * B,), jnp.float32),
    mesh=plsc.VectorSubcoreMesh(core_axis_name="c", subcore_axis_name="s"),
    scratch_types=[
        pltpu.VMEM((2, B), jnp.int32),
        pltpu.VMEM((2, B), jnp.float32),
        pltpu.VMEM((B,), jnp.float32),
        pltpu.SemaphoreType.DMA,
        pltpu.SemaphoreType.DMA,
    ],
    compiler_params=pltpu.CompilerParams(use_tc_tiling_on_sc=False),
)
def _emb_kernel(tab_hbm, idx_hbm, num_hbm, out_hbm, idx2, col2, numv, gsem, wsem):
    wid = lax.axis_index("s") * NC + lax.axis_index("c")

    def prep(c):
        j = wid * CPW + c
        f = lax.div(j, D)
        pltpu.sync_copy(idx_hbm.at[pl.ds(f * B, B)], idx2.at[c % 2])
        off = j * V

        def addo(p, carry):
            idx2[c % 2, pl.ds(p * L, L)] = idx2[c % 2, pl.ds(p * L, L)] + off
            return carry

        lax.fori_loop(0, B // L, addo, 0)

    def fire(c):
        def go(k, carry):
            pltpu.make_async_copy(
                tab_hbm.at[idx2.at[c % 2, pl.ds(k * SL, SL)]],
                col2.at[c % 2, pl.ds(k * SL, SL)],
                gsem,
            ).start()
            return carry

        lax.fori_loop(0, NST, go, 0)

    def drain(c):
        def go(k, carry):
            pltpu.make_async_copy(
                tab_hbm.at[idx2.at[c % 2, pl.ds(k * SL, SL)]],
                col2.at[c % 2, pl.ds(k * SL, SL)],
                gsem,
            ).wait()
            return carry

        lax.fori_loop(0, NST, go, 0)

    def wb(c):
        j = wid * CPW + c
        return pltpu.make_async_copy(
            col2.at[c % 2],
            out_hbm.at[pl.ds((NUM + j) * B, B)],
            wsem,
        )

    prep(0)
    fire(0)

    def body(c, carry):
        @pl.when(c + 1 < CPW)
        def _():
            prep(c + 1)

            @pl.when(c >= 1)
            def _():
                wb(c - 1).wait()

            fire(c + 1)

        drain(c)
        wb(c).start()
        return carry

    lax.fori_loop(0, CPW, body, 0)
    wb(CPW - 2).wait()
    wb(CPW - 1).wait()

    @pl.when(wid < NUM)
    def _():
        pltpu.sync_copy(num_hbm.at[pl.ds(wid * B, B)], numv)
        pltpu.sync_copy(numv, out_hbm.at[pl.ds(wid * B, B)])


def kernel(indices, numerical, tables):
    tab = jnp.transpose(tables, (0, 2, 1)).reshape(F * D * V)
    idx_t = jnp.transpose(indices).reshape(F * B)
    num_t = jnp.transpose(numerical).reshape(NUM * B)
    out = _emb_kernel(tab, idx_t, num_t)
    return jnp.transpose(out.reshape(OWP, B))[:, :OW]
